# baseline (speedup 1.0000x reference)
import functools

import jax
import jax.numpy as jnp
from jax import lax
from jax.experimental import pallas as pl
from jax.experimental.pallas import tpu as pltpu
from jax.experimental.pallas import tpu_sc as plsc


def _gather_kernel(B, F, V, D):
    info = plsc.get_sparse_core_info()
    NC, NS, L = info.num_cores, info.num_subcores, info.num_lanes
    NW = NC * NS
    assert B % NW == 0
    bpw = B // NW
    RW = 128 // D

    mesh = plsc.VectorSubcoreMesh(core_axis_name="c", subcore_axis_name="s")

    @functools.partial(
        pl.kernel,
        mesh=mesh,
        out_type=jax.ShapeDtypeStruct((B, F, D), jnp.float32),
        scratch_types=[
            pltpu.VMEM((bpw,), jnp.int32),
            pltpu.VMEM((bpw + L,), jnp.int32),
            pltpu.VMEM((bpw, 128), jnp.float32),
            pltpu.VMEM((bpw, D), jnp.float32),
            pltpu.SemaphoreType.DMA,
        ],
    )
    def k(idx_hbm, tab_hbm, out_hbm, ids_v, offs_v, rows_v, ext_v, sem):
        wid = lax.axis_index("s") * NC + lax.axis_index("c")
        b0 = wid * bpw

        def field(f, carry):
            pltpu.sync_copy(idx_hbm.at[f, pl.ds(b0, bpw)], ids_v)

            def mkids(i, c):
                sl = pl.ds(i * L, L)
                r = ids_v[sl] + f * V
                ids_v[sl] = lax.shift_right_logical(r, 2)
                offs_v[sl] = lax.shift_left(jnp.bitwise_and(r, RW - 1), 5)
                return c

            lax.fori_loop(0, bpw // L, mkids, 0)
            pltpu.async_copy(tab_hbm.at[ids_v], rows_v, sem).wait()

            def ext(j, c):
                off = offs_v[pl.ds(j, L)][0]
                for h in range(D // L):
                    ext_v[j, pl.ds(h * L, L)] = rows_v[j, pl.ds(off + h * L, L)]
                return c

            lax.fori_loop(0, bpw, ext, 0)
            pltpu.sync_copy(ext_v, out_hbm.at[pl.ds(b0, bpw), f, :])
            return carry

        lax.fori_loop(0, F, field, 0)

    return k


def kernel(inputs, tables):
    B, F = inputs.shape
    _, V, D = tables.shape
    idx_t = inputs.T
    tab128 = tables.reshape((F * V * D) // 128, 128)
    return _gather_kernel(B, F, V, D)(idx_t, tab128)


# speedup vs baseline: 1.2530x; 1.2530x over previous
import functools

import jax
import jax.numpy as jnp
from jax import lax
from jax.experimental import pallas as pl
from jax.experimental.pallas import tpu as pltpu
from jax.experimental.pallas import tpu_sc as plsc


def _gather_kernel(B, F, V, D):
    info = plsc.get_sparse_core_info()
    NC, NS, L = info.num_cores, info.num_subcores, info.num_lanes
    NW = NC * NS
    assert B % NW == 0
    bpw = B // NW
    C = 64

    mesh = plsc.VectorSubcoreMesh(core_axis_name="c", subcore_axis_name="s")

    @functools.partial(
        pl.kernel,
        mesh=mesh,
        out_type=jax.ShapeDtypeStruct((B, F, D), jnp.float32),
        scratch_types=[
            pltpu.VMEM((bpw,), jnp.int32),
            pltpu.VMEM((bpw + L,), jnp.int32),
            pltpu.VMEM((C, 8, D), jnp.float32),
            pltpu.VMEM((bpw, D), jnp.float32),
            pltpu.SemaphoreType.DMA,
            pltpu.SemaphoreType.DMA,
        ],
    )
    def k(idx_hbm, tab_hbm, out_hbm, idx_v, sidx_v, grp_v, ext_v, sem_g, sem_w):
        wid = lax.axis_index("s") * NC + lax.axis_index("c")
        b0 = wid * bpw

        def field(f, carry):
            pltpu.sync_copy(idx_hbm.at[f, pl.ds(b0, bpw)], idx_v)

            def stage(i, c):
                sl = pl.ds(i * L, L)
                sidx_v[sl] = idx_v[sl]
                return c

            lax.fori_loop(0, bpw // L, stage, 0)

            def chunk(q, cc):
                q0 = q * C

                def fire(j, c):
                    v = sidx_v[pl.ds(q0 + j, L)][0]
                    va = pl.multiple_of(v & ~7, 8)
                    pltpu.async_copy(
                        tab_hbm.at[f, pl.ds(va, 8), :], grp_v.at[j], sem_g
                    )
                    return c

                lax.fori_loop(0, C, fire, 0)

                def drain(j, c):
                    pltpu.make_async_copy(
                        tab_hbm.at[f, pl.ds(0, 8), :], grp_v.at[j], sem_g
                    ).wait()
                    return c

                lax.fori_loop(0, C, drain, 0)

                def ext(j, c):
                    s = sidx_v[pl.ds(q0 + j, L)][0] & 7
                    for h in range(D // L):
                        ext_v[q0 + j, pl.ds(h * L, L)] = grp_v[j, s, pl.ds(h * L, L)]
                    return c

                lax.fori_loop(0, C, ext, 0)
                return cc

            lax.fori_loop(0, bpw // C, chunk, 0)
            pltpu.sync_copy(ext_v, out_hbm.at[pl.ds(b0, bpw), f, :])
            return carry

        lax.fori_loop(0, F, field, 0)

    return k


def kernel(inputs, tables):
    B, F = inputs.shape
    _, V, D = tables.shape
    idx_t = inputs.T
    return _gather_kernel(B, F, V, D)(idx_t, tables)


# speedup vs baseline: 2.1521x; 1.7176x over previous
import functools

import jax
import jax.numpy as jnp
from jax import lax
from jax.experimental import pallas as pl
from jax.experimental.pallas import tpu as pltpu
from jax.experimental.pallas import tpu_sc as plsc


def _gather_kernel(B, F, V, D):
    info = plsc.get_sparse_core_info()
    NC, NS, L = info.num_cores, info.num_subcores, info.num_lanes
    NW = NC * NS
    assert B % NW == 0
    bpw = B // NW
    C = 64

    mesh = plsc.VectorSubcoreMesh(core_axis_name="c", subcore_axis_name="s")

    @functools.partial(
        pl.kernel,
        mesh=mesh,
        out_type=jax.ShapeDtypeStruct((B, F, D), jnp.float32),
        scratch_types=[
            pltpu.VMEM((bpw,), jnp.int32),
            pltpu.VMEM((bpw + L,), jnp.int32),
            pltpu.VMEM((C, 8, D), jnp.float32),
            pltpu.VMEM((bpw, D), jnp.float32),
            pltpu.SemaphoreType.DMA,
            pltpu.SemaphoreType.DMA,
        ],
    )
    def k(idx_hbm, tab_hbm, out_hbm, idx_v, sidx_v, grp_v, ext_v, sem_g, sem_w):
        wid = lax.axis_index("s") * NC + lax.axis_index("c")
        b0 = wid * bpw

        def field(f, carry):
            pltpu.sync_copy(idx_hbm.at[f, pl.ds(b0, bpw)], idx_v)

            def stage(i, c):
                sl = pl.ds(i * L, L)
                sidx_v[sl] = idx_v[sl]
                return c

            lax.fori_loop(0, bpw // L, stage, 0)

            def chunk(q, cc):
                q0 = q * C

                def fire(j, c):
                    v = sidx_v[pl.ds(q0 + j, L)][0]
                    vg = lax.shift_right_logical(v, 3)
                    pltpu.async_copy(
                        tab_hbm.at[f, vg, :, :], grp_v.at[j], sem_g
                    )
                    return c

                lax.fori_loop(0, C, fire, 0)

                def drain(j, c):
                    pltpu.make_async_copy(
                        tab_hbm.at[f, 0, :, :], grp_v.at[j], sem_g
                    ).wait()
                    return c

                lax.fori_loop(0, C, drain, 0)

                def ext(j, c):
                    s = sidx_v[pl.ds(q0 + j, L)][0] & 7
                    for h in range(D // L):
                        ext_v[q0 + j, pl.ds(h * L, L)] = grp_v[j, s, pl.ds(h * L, L)]
                    return c

                lax.fori_loop(0, C, ext, 0)
                return cc

            lax.fori_loop(0, bpw // C, chunk, 0)
            pltpu.sync_copy(ext_v, out_hbm.at[pl.ds(b0, bpw), f, :])
            return carry

        lax.fori_loop(0, F, field, 0)

    return k


def kernel(inputs, tables):
    B, F = inputs.shape
    _, V, D = tables.shape
    idx_t = inputs.T
    tab4 = tables.reshape(F, V // 8, 8, D)
    return _gather_kernel(B, F, V, D)(idx_t, tab4)


# speedup vs baseline: 2.3422x; 1.0883x over previous
import functools

import jax
import jax.numpy as jnp
from jax import lax
from jax.experimental import pallas as pl
from jax.experimental.pallas import tpu as pltpu
from jax.experimental.pallas import tpu_sc as plsc


def _gather_kernel(B, F, V, D):
    info = plsc.get_sparse_core_info()
    NC, NS, L = info.num_cores, info.num_subcores, info.num_lanes
    NW = NC * NS
    assert B % NW == 0
    bpw = B // NW
    C = 32
    NCHUNK = bpw // C

    mesh = plsc.VectorSubcoreMesh(core_axis_name="c", subcore_axis_name="s")

    @functools.partial(
        pl.kernel,
        mesh=mesh,
        out_type=jax.ShapeDtypeStruct((B, F, D), jnp.float32),
        scratch_types=[
            pltpu.VMEM((F, bpw), jnp.int32),
            pltpu.VMEM((F * bpw + L,), jnp.int32),
            pltpu.VMEM((C, 8, D), jnp.float32),
            pltpu.VMEM((C, 8, D), jnp.float32),
            pltpu.VMEM((C, D), jnp.float32),
            pltpu.VMEM((C, D), jnp.float32),
            pltpu.SemaphoreType.DMA,
            pltpu.SemaphoreType.DMA,
            pltpu.SemaphoreType.DMA,
            pltpu.SemaphoreType.DMA,
        ],
    )
    def k(idx_hbm, tab_hbm, out_hbm, idx_v, sidx_v, gA, gB, eA, eB,
          sgA, sgB, swA, swB):
        wid = lax.axis_index("s") * NC + lax.axis_index("c")
        b0 = wid * bpw

        pltpu.sync_copy(idx_hbm.at[:, pl.ds(b0, bpw)], idx_v)

        def stage(i, c):
            f = lax.shift_right_logical(i, 3)
            bb = jnp.bitwise_and(i, 7) * L
            sidx_v[pl.ds(i * L, L)] = idx_v[f, pl.ds(bb, L)]
            return c

        lax.fori_loop(0, (F * bpw) // L, stage, 0)

        grps = (gA, gB)
        exts = (eA, eB)
        sgs = (sgA, sgB)
        sws = (swA, swB)

        def fire(f, q0, grp, sem):
            def go(j, c):
                v = sidx_v[pl.ds(f * bpw + q0 + j, L)][0]
                vg = lax.shift_right_logical(v, 3)
                pltpu.async_copy(tab_hbm.at[f, vg, :, :], grp.at[j], sem)
                return c

            lax.fori_loop(0, C, go, 0)

        def drain_g(grp, sem):
            def go(j, c):
                pltpu.make_async_copy(
                    tab_hbm.at[0, 0, :, :], grp.at[j], sem
                ).wait()
                return c

            lax.fori_loop(0, C, go, 0)

        def extract(f, q0, grp, ext):
            def go(j, c):
                s = jnp.bitwise_and(sidx_v[pl.ds(f * bpw + q0 + j, L)][0], 7)
                for h in range(D // L):
                    ext[j, pl.ds(h * L, L)] = grp[j, s, pl.ds(h * L, L)]
                return c

            lax.fori_loop(0, C, go, 0)

        def out_write(f, q0, ext, sem):
            pltpu.async_copy(ext, out_hbm.at[pl.ds(b0 + q0, C), f, :], sem)

        def drain_w(ext, sem):
            pltpu.make_async_copy(
                ext, out_hbm.at[pl.ds(b0, C), 0, :], sem
            ).wait()

        def field(fi, carry):
            fire(fi, 0, grps[0], sgs[0])
            for q in range(NCHUNK):
                p = q % 2
                if q + 1 < NCHUNK:
                    fire(fi, (q + 1) * C, grps[(q + 1) % 2], sgs[(q + 1) % 2])
                drain_g(grps[p], sgs[p])
                if q >= 2:
                    drain_w(exts[p], sws[p])
                else:
                    @pl.when(fi > 0)
                    def _():
                        drain_w(exts[p], sws[p])
                extract(fi, q * C, grps[p], exts[p])
                out_write(fi, q * C, exts[p], sws[p])
            return carry

        lax.fori_loop(0, F, field, 0)
        drain_w(exts[0], sws[0])
        drain_w(exts[1], sws[1])

    return k


def kernel(inputs, tables):
    B, F = inputs.shape
    _, V, D = tables.shape
    idx_t = inputs.T
    tab4 = tables.reshape(F, V // 8, 8, D)
    return _gather_kernel(B, F, V, D)(idx_t, tab4)


# speedup vs baseline: 2.4211x; 1.0337x over previous
import functools

import jax
import jax.numpy as jnp
from jax import lax
from jax.experimental import pallas as pl
from jax.experimental.pallas import tpu as pltpu
from jax.experimental.pallas import tpu_sc as plsc


def _gather_kernel(B, F, V, D):
    info = plsc.get_sparse_core_info()
    NC, NS, L = info.num_cores, info.num_subcores, info.num_lanes
    NW = NC * NS
    assert B % NW == 0
    bpw = B // NW
    C = 32
    NCHUNK = bpw // C

    mesh = plsc.VectorSubcoreMesh(core_axis_name="c", subcore_axis_name="s")

    @functools.partial(
        pl.kernel,
        mesh=mesh,
        out_type=jax.ShapeDtypeStruct((B, F, D), jnp.float32),
        scratch_types=[
            pltpu.VMEM((F, bpw), jnp.int32),
            pltpu.VMEM((F * bpw + L,), jnp.int32),
            pltpu.VMEM((C, 8, D), jnp.float32),
            pltpu.VMEM((C, 8, D), jnp.float32),
            pltpu.VMEM((C, D), jnp.float32),
            pltpu.VMEM((C, D), jnp.float32),
            pltpu.SemaphoreType.DMA,
            pltpu.SemaphoreType.DMA,
            pltpu.SemaphoreType.DMA,
            pltpu.SemaphoreType.DMA,
        ],
    )
    def k(idx_hbm, tab_hbm, out_hbm, idx_v, sidx_v, gA, gB, eA, eB,
          sgA, sgB, swA, swB):
        wid = lax.axis_index("s") * NC + lax.axis_index("c")
        b0 = wid * bpw

        pltpu.sync_copy(idx_hbm.at[:, pl.ds(b0, bpw)], idx_v)

        def stage(i, c):
            f = lax.shift_right_logical(i, 3)
            bb = jnp.bitwise_and(i, 7) * L
            sidx_v[pl.ds(i * L, L)] = idx_v[f, pl.ds(bb, L)]
            return c

        lax.fori_loop(0, (F * bpw) // L, stage, 0)

        grps = (gA, gB)
        exts = (eA, eB)
        sgs = (sgA, sgB)
        sws = (swA, swB)

        def fire(f, q0, grp, sem):
            def go(i, c):
                w = sidx_v[pl.ds(f * bpw + q0 + i * L, L)]
                for kk in range(L):
                    vg = lax.shift_right_logical(w[kk], 3)
                    pltpu.async_copy(
                        tab_hbm.at[f, vg, :, :], grp.at[i * L + kk], sem
                    )
                return c

            lax.fori_loop(0, C // L, go, 0)

        def drain_g(grp, sem):
            pltpu.make_async_copy(
                tab_hbm.at[0, pl.ds(0, C), :, :], grp, sem
            ).wait()

        def extract(f, q0, grp, ext):
            def go(i, c):
                w = sidx_v[pl.ds(f * bpw + q0 + i * L, L)]
                for kk in range(L):
                    j = i * L + kk
                    s = jnp.bitwise_and(w[kk], 7)
                    for h in range(D // L):
                        ext[j, pl.ds(h * L, L)] = grp[j, s, pl.ds(h * L, L)]
                return c

            lax.fori_loop(0, C // L, go, 0)

        def out_write(f, q0, ext, sem):
            pltpu.async_copy(ext, out_hbm.at[pl.ds(b0 + q0, C), f, :], sem)

        def drain_w(ext, sem):
            pltpu.make_async_copy(
                ext, out_hbm.at[pl.ds(b0, C), 0, :], sem
            ).wait()

        def field(fi, carry):
            fire(fi, 0, grps[0], sgs[0])
            for q in range(NCHUNK):
                p = q % 2
                if q + 1 < NCHUNK:
                    fire(fi, (q + 1) * C, grps[(q + 1) % 2], sgs[(q + 1) % 2])
                drain_g(grps[p], sgs[p])
                if q >= 2:
                    drain_w(exts[p], sws[p])
                else:
                    @pl.when(fi > 0)
                    def _():
                        drain_w(exts[p], sws[p])
                extract(fi, q * C, grps[p], exts[p])
                out_write(fi, q * C, exts[p], sws[p])
            return carry

        lax.fori_loop(0, F, field, 0)
        drain_w(exts[0], sws[0])
        drain_w(exts[1], sws[1])

    return k


def kernel(inputs, tables):
    B, F = inputs.shape
    _, V, D = tables.shape
    idx_t = inputs.T
    tab4 = tables.reshape(F, V // 8, 8, D)
    return _gather_kernel(B, F, V, D)(idx_t, tab4)


# speedup vs baseline: 2.4597x; 1.0160x over previous
import functools

import jax
import jax.numpy as jnp
from jax import lax
from jax.experimental import pallas as pl
from jax.experimental.pallas import tpu as pltpu
from jax.experimental.pallas import tpu_sc as plsc


def _gather_kernel(B, F, V, D):
    info = plsc.get_sparse_core_info()
    NC, NS, L = info.num_cores, info.num_subcores, info.num_lanes
    NW = NC * NS
    assert B % NW == 0
    bpw = B // NW
    C = 16
    NCHUNK = bpw // C

    mesh = plsc.VectorSubcoreMesh(core_axis_name="c", subcore_axis_name="s")

    @functools.partial(
        pl.kernel,
        mesh=mesh,
        out_type=jax.ShapeDtypeStruct((B, F, D), jnp.float32),
        scratch_types=[
            pltpu.VMEM((F, bpw), jnp.int32),
            pltpu.VMEM((F * bpw + L,), jnp.int32),
        ] + [pltpu.VMEM((C, 8, D), jnp.float32)] * 4
          + [pltpu.VMEM((C, D), jnp.float32)] * 4
          + [pltpu.SemaphoreType.DMA] * 8,
    )
    def k(idx_hbm, tab_hbm, out_hbm, idx_v, sidx_v,
          g0, g1, g2, g3, e0, e1, e2, e3,
          sg0, sg1, sg2, sg3, sw0, sw1, sw2, sw3):
        wid = lax.axis_index("s") * NC + lax.axis_index("c")
        b0 = wid * bpw

        pltpu.sync_copy(idx_hbm.at[:, pl.ds(b0, bpw)], idx_v)

        def stage(i, c):
            f = lax.shift_right_logical(i, 3)
            bb = jnp.bitwise_and(i, 7) * L
            sidx_v[pl.ds(i * L, L)] = idx_v[f, pl.ds(bb, L)]
            return c

        lax.fori_loop(0, (F * bpw) // L, stage, 0)

        grps = (g0, g1, g2, g3)
        exts = (e0, e1, e2, e3)
        sgs = (sg0, sg1, sg2, sg3)
        sws = (sw0, sw1, sw2, sw3)

        def fire(f, q0, grp, sem):
            def go(i, c):
                w = sidx_v[pl.ds(f * bpw + q0 + i * L, L)]
                for kk in range(L):
                    vg = lax.shift_right_logical(w[kk], 3)
                    pltpu.async_copy(
                        tab_hbm.at[f, vg, :, :], grp.at[i * L + kk], sem
                    )
                return c

            lax.fori_loop(0, C // L, go, 0)

        def drain_g(grp, sem):
            pltpu.make_async_copy(
                tab_hbm.at[0, pl.ds(0, C), :, :], grp, sem
            ).wait()

        def extract(f, q0, grp, ext):
            def go(i, c):
                w = sidx_v[pl.ds(f * bpw + q0 + i * L, L)]
                for kk in range(L):
                    j = i * L + kk
                    s = jnp.bitwise_and(w[kk], 7)
                    for h in range(D // L):
                        ext[j, pl.ds(h * L, L)] = grp[j, s, pl.ds(h * L, L)]
                return c

            lax.fori_loop(0, C // L, go, 0)

        def out_write(f, q0, ext, sem):
            pltpu.async_copy(ext, out_hbm.at[pl.ds(b0 + q0, C), f, :], sem)

        def drain_w(ext, sem):
            pltpu.make_async_copy(
                ext, out_hbm.at[pl.ds(b0, C), 0, :], sem
            ).wait()

        NB = 4

        def field(fi, carry):
            for q in range(NB - 1):
                fire(fi, q * C, grps[q], sgs[q])
            for q in range(NCHUNK):
                p = q % NB
                if q + NB - 1 < NCHUNK:
                    qq = q + NB - 1
                    fire(fi, qq * C, grps[qq % NB], sgs[qq % NB])
                drain_g(grps[p], sgs[p])
                if q >= NB:
                    drain_w(exts[p], sws[p])
                else:
                    @pl.when(fi > 0)
                    def _():
                        drain_w(exts[p], sws[p])
                extract(fi, q * C, grps[p], exts[p])
                out_write(fi, q * C, exts[p], sws[p])
            return carry

        lax.fori_loop(0, F, field, 0)
        for q in range(NB):
            drain_w(exts[q], sws[q])

    return k


def kernel(inputs, tables):
    B, F = inputs.shape
    _, V, D = tables.shape
    idx_t = inputs.T
    tab4 = tables.reshape(F, V // 8, 8, D)
    return _gather_kernel(B, F, V, D)(idx_t, tab4)
